# baseline (speedup 1.0000x reference)
import functools

import jax
import jax.numpy as jnp
import numpy as np
from jax.experimental import pallas as pl
from jax.experimental.pallas import tpu as pltpu

B = 8192
L1 = 2048
L2 = 16
L3 = 32
E = 16
RF = 32
BB = 1024
NSTEPS = B // BB

_DGT = (((1,), (1,)), ((), ()))


def _dgt(a, b):
    return jax.lax.dot_general(a, b, _DGT, preferred_element_type=jnp.float32)


def _fused_kernel(x_ref, w1a_ref, w1b_ref, q_ref,
                  l3x_ref, frac_ref, avg_ref, aux_ref, z_ref, ent_ref,
                  top1_ref, std_ref, acc_ref):
    i = pl.program_id(0)
    xb = x_ref[...]
    q = q_ref[...]
    biases = q[528:529, :]

    logits = (
        _dgt(xb[:, :128], q[530:546, 0:128])
        + _dgt(xb[:, 1024:1152], q[530:546, 128:256])
        + biases[:, 272:288]
    )
    m = jnp.max(logits, axis=1, keepdims=True)
    ex = jnp.exp(logits - m)
    se = jnp.sum(ex, axis=1, keepdims=True)
    probs = ex / se
    lse = jnp.log(se) + m

    iota = jax.lax.broadcasted_iota(jnp.int32, logits.shape, 1)
    idx = jnp.min(jnp.where(logits == m, iota, E), axis=1, keepdims=True)
    onehot = (iota == idx).astype(jnp.float32)

    probs_sum = jnp.sum(probs, axis=0, keepdims=True)
    counts = jnp.sum(onehot, axis=0, keepdims=True)
    z_part = jnp.sum(lse * lse, axis=0, keepdims=True)
    ent_tok = jnp.sum(-(probs * jnp.log(jnp.maximum(probs, 1e-9))),
                      axis=1, keepdims=True)
    ent_part = jnp.sum(ent_tok, axis=0, keepdims=True)
    top1_part = jnp.sum(jnp.max(probs, axis=1, keepdims=True),
                        axis=0, keepdims=True)

    l1a = jnp.dot(xb, w1a_ref[...],
                  preferred_element_type=jnp.float32) + biases[:, :256]
    l1o = jnp.dot(xb, w1b_ref[...],
                  preferred_element_type=jnp.float32) + biases[:, 256:272]
    sq = jnp.clip(l1a * l1a * (255.0 / 256.0), 0.0, 1.0)
    lin = jnp.clip(l1a, 0.0, 1.0)
    b2 = q[529:530, :]
    l2x0 = jnp.clip(
        jnp.dot(sq[:, :128], q[0:128, 0:256], preferred_element_type=jnp.float32)
        + jnp.dot(lin[:, :128], q[256:384, 0:256], preferred_element_type=jnp.float32)
        + b2[:, :256], 0.0, 1.0)
    l2x1 = jnp.clip(
        jnp.dot(sq[:, 128:], q[128:256, 256:512], preferred_element_type=jnp.float32)
        + jnp.dot(lin[:, 128:], q[384:512, 256:512], preferred_element_type=jnp.float32)
        + b2[:, 256:], 0.0, 1.0)
    y = (_dgt(l2x0, q[512:528, 0:256])
         + _dgt(l2x1, q[512:528, 256:512])
         + biases[:, 288:304] + l1o)

    mean_e = jnp.mean(y, axis=1, keepdims=True)
    var = jnp.mean(y * y, axis=1, keepdims=True) - mean_e * mean_e
    stdv = jnp.sqrt(jnp.maximum(var, 0.0))
    std_part = jnp.sum(stdv, axis=0, keepdims=True)

    l3x_ref[...] = jnp.sum(y * onehot, axis=1, keepdims=True)

    bc = lambda v: jnp.broadcast_to(v, (1, E))
    part = jnp.concatenate(
        [counts, probs_sum, bc(z_part), bc(ent_part), bc(top1_part),
         bc(std_part), jnp.zeros((2, E), jnp.float32)], axis=0)

    @pl.when(i == 0)
    def _init():
        acc_ref[...] = part

    @pl.when(i > 0)
    def _acc():
        acc_ref[...] += part

    @pl.when(i == NSTEPS - 1)
    def _finalize():
        st = acc_ref[...] / float(B)
        fr = st[0:1, :]
        av = st[1:2, :]
        frac_ref[...] = fr
        avg_ref[...] = av
        aux_ref[...] = float(E) * jnp.sum(fr * av, axis=1, keepdims=True)
        z_ref[...] = st[2:3, 0:1]
        ent_ref[...] = st[3:4, 0:1] / float(np.log(E))
        top1_ref[...] = st[4:5, 0:1]
        std_ref[...] = st[5:6, 0:1]


@jax.jit
def kernel(x, router_w, router_b, l1_w, l1_b, l2_w, l2_b, out_w, out_b):
    f32 = jnp.float32
    eye = jnp.eye(E, dtype=f32)

    w1a = l1_w[:, :L2, :].reshape(E * L2, L1).T
    w1b = l1_w[:, L2, :].T

    bda = jnp.einsum('ef,eoi->eifo', eye,
                     l2_w[:, :, :L2]).reshape(E * L2, E * L3)
    bdb = jnp.einsum('ef,eoi->eifo', eye,
                     l2_w[:, :, L2:]).reshape(E * L2, E * L3)
    swt = jnp.einsum('ef,eo->feo', eye, out_w[:, 0, :]).reshape(E, E * L3)
    brow = jnp.concatenate(
        [l1_b[:, :L2].reshape(1, E * L2), l1_b[:, L2].reshape(1, E),
         router_b.reshape(1, E), out_b[:, 0].reshape(1, E),
         jnp.zeros((1, E * L3 - E * L2 - 3 * E), f32)], axis=1)
    rtr = jnp.zeros((E, E * L3), f32)
    rtr = rtr.at[:, :RF].set(router_w[:, :RF])
    rtr = rtr.at[:, 128:128 + RF].set(router_w[:, RF:])
    q = jnp.concatenate(
        [bda, bdb, swt, brow, l2_b.reshape(1, E * L3), rtr,
         jnp.zeros((6, E * L3), f32)], axis=0)

    full = lambda shape: pl.BlockSpec(shape, lambda i: (0, 0))
    out_shapes = (
        jax.ShapeDtypeStruct((B, 1), f32),
        jax.ShapeDtypeStruct((1, E), f32),
        jax.ShapeDtypeStruct((1, E), f32),
        jax.ShapeDtypeStruct((1, 1), f32),
        jax.ShapeDtypeStruct((1, 1), f32),
        jax.ShapeDtypeStruct((1, 1), f32),
        jax.ShapeDtypeStruct((1, 1), f32),
        jax.ShapeDtypeStruct((1, 1), f32),
    )
    outs = pl.pallas_call(
        _fused_kernel,
        grid=(NSTEPS,),
        in_specs=[
            pl.BlockSpec((BB, L1), lambda i: (i, 0)),
            full((L1, E * L2)), full((L1, E)),
            full((552, E * L3)),
        ],
        out_specs=(
            pl.BlockSpec((BB, 1), lambda i: (i, 0)),
            full((1, E)), full((1, E)), full((1, 1)), full((1, 1)),
            full((1, 1)), full((1, 1)), full((1, 1)),
        ),
        out_shape=out_shapes,
        scratch_shapes=[pltpu.VMEM((8, E), jnp.float32)],
        compiler_params=pltpu.CompilerParams(
            dimension_semantics=("arbitrary",)),
    )(x, w1a, w1b, q)

    l3x, frac, avg, aux, z, ent, top1, std = outs
    return (l3x, aux[0, 0], z[0, 0], frac[0], avg[0],
            ent[0, 0], top1[0, 0], std[0, 0])


# speedup vs baseline: 1.0154x; 1.0154x over previous
import functools

import jax
import jax.numpy as jnp
import numpy as np
from jax.experimental import pallas as pl
from jax.experimental.pallas import tpu as pltpu

B = 8192
L1 = 2048
L2 = 16
L3 = 32
E = 16
RF = 32
BB = 512
NSTEPS = B // BB


def _fused_kernel(x_ref, rwa_ref, rwb_ref, rb_ref, w1a_ref, b1a_ref,
                  w1b_ref, b1b_ref, bda_ref, bdb_ref, b2_ref, sw_ref, ob_ref,
                  l3x_ref, frac_ref, avg_ref, aux_ref, z_ref, ent_ref,
                  top1_ref, std_ref, acc_ref):
    i = pl.program_id(0)
    xb = x_ref[...]

    logits = (
        jnp.dot(xb[:, :128], rwa_ref[...], preferred_element_type=jnp.float32)
        + jnp.dot(xb[:, 1024:1152], rwb_ref[...],
                  preferred_element_type=jnp.float32)
        + rb_ref[...]
    )
    m = jnp.max(logits, axis=1, keepdims=True)
    ex = jnp.exp(logits - m)
    se = jnp.sum(ex, axis=1, keepdims=True)
    probs = ex / se
    lse = jnp.log(se) + m

    iota = jax.lax.broadcasted_iota(jnp.int32, logits.shape, 1)
    idx = jnp.min(jnp.where(logits == m, iota, E), axis=1, keepdims=True)
    onehot = (iota == idx).astype(jnp.float32)

    probs_sum = jnp.sum(probs, axis=0, keepdims=True)
    counts = jnp.sum(onehot, axis=0, keepdims=True)
    z_part = jnp.sum(lse * lse, axis=0, keepdims=True)
    ent_tok = jnp.sum(-(probs * jnp.log(jnp.maximum(probs, 1e-9))),
                      axis=1, keepdims=True)
    ent_part = jnp.sum(ent_tok, axis=0, keepdims=True)
    top1_part = jnp.sum(jnp.max(probs, axis=1, keepdims=True),
                        axis=0, keepdims=True)

    l1a = jnp.dot(xb, w1a_ref[...],
                  preferred_element_type=jnp.float32) + b1a_ref[...]
    l1o = jnp.dot(xb, w1b_ref[...],
                  preferred_element_type=jnp.float32) + b1b_ref[...]
    sq = jnp.clip(l1a * l1a * (255.0 / 256.0), 0.0, 1.0)
    lin = jnp.clip(l1a, 0.0, 1.0)
    b2 = b2_ref[...]
    bda = bda_ref[...]
    bdb = bdb_ref[...]
    sw = sw_ref[...]
    l2x0 = jnp.clip(
        jnp.dot(sq[:, :128], bda[:128, :256], preferred_element_type=jnp.float32)
        + jnp.dot(lin[:, :128], bdb[:128, :256], preferred_element_type=jnp.float32)
        + b2[:, :256], 0.0, 1.0)
    l2x1 = jnp.clip(
        jnp.dot(sq[:, 128:], bda[128:, 256:], preferred_element_type=jnp.float32)
        + jnp.dot(lin[:, 128:], bdb[128:, 256:], preferred_element_type=jnp.float32)
        + b2[:, 256:], 0.0, 1.0)
    y = (jnp.dot(l2x0, sw[:256, :], preferred_element_type=jnp.float32)
         + jnp.dot(l2x1, sw[256:, :], preferred_element_type=jnp.float32)
         + ob_ref[...] + l1o)

    mean_e = jnp.mean(y, axis=1, keepdims=True)
    var = jnp.mean(y * y, axis=1, keepdims=True) - mean_e * mean_e
    stdv = jnp.sqrt(jnp.maximum(var, 0.0))
    std_part = jnp.sum(stdv, axis=0, keepdims=True)

    l3x_ref[...] = jnp.sum(y * onehot, axis=1, keepdims=True)

    bc = lambda v: jnp.broadcast_to(v, (1, E))
    part = jnp.concatenate(
        [counts, probs_sum, bc(z_part), bc(ent_part), bc(top1_part),
         bc(std_part), jnp.zeros((2, E), jnp.float32)], axis=0)

    @pl.when(i == 0)
    def _init():
        acc_ref[...] = part

    @pl.when(i > 0)
    def _acc():
        acc_ref[...] += part

    @pl.when(i == NSTEPS - 1)
    def _finalize():
        st = acc_ref[...] / float(B)
        fr = st[0:1, :]
        av = st[1:2, :]
        frac_ref[...] = fr
        avg_ref[...] = av
        aux_ref[...] = float(E) * jnp.sum(fr * av, axis=1, keepdims=True)
        z_ref[...] = st[2:3, 0:1]
        ent_ref[...] = st[3:4, 0:1] / float(np.log(E))
        top1_ref[...] = st[4:5, 0:1]
        std_ref[...] = st[5:6, 0:1]


@jax.jit
def kernel(x, router_w, router_b, l1_w, l1_b, l2_w, l2_b, out_w, out_b):
    f32 = jnp.float32
    eye = jnp.eye(E, dtype=f32)

    rwa = jnp.zeros((128, E), f32).at[:RF, :].set(router_w[:, :RF].T)
    rwb = jnp.zeros((128, E), f32).at[:RF, :].set(router_w[:, RF:].T)
    rb = router_b.reshape(1, E)

    w1a = l1_w[:, :L2, :].reshape(E * L2, L1).T
    b1a = l1_b[:, :L2].reshape(1, E * L2)
    w1b = l1_w[:, L2, :].T
    b1b = l1_b[:, L2].reshape(1, E)

    bda = jnp.einsum('ef,eoi->eifo', eye,
                     l2_w[:, :, :L2]).reshape(E * L2, E * L3)
    bdb = jnp.einsum('ef,eoi->eifo', eye,
                     l2_w[:, :, L2:]).reshape(E * L2, E * L3)
    b2 = l2_b.reshape(1, E * L3)

    sw = jnp.einsum('ef,eo->eof', eye, out_w[:, 0, :]).reshape(E * L3, E)
    ob = out_b[:, 0].reshape(1, E)

    full = lambda shape: pl.BlockSpec(shape, lambda i: (0, 0))
    out_shapes = (
        jax.ShapeDtypeStruct((B, 1), f32),
        jax.ShapeDtypeStruct((1, E), f32),
        jax.ShapeDtypeStruct((1, E), f32),
        jax.ShapeDtypeStruct((1, 1), f32),
        jax.ShapeDtypeStruct((1, 1), f32),
        jax.ShapeDtypeStruct((1, 1), f32),
        jax.ShapeDtypeStruct((1, 1), f32),
        jax.ShapeDtypeStruct((1, 1), f32),
    )
    outs = pl.pallas_call(
        _fused_kernel,
        grid=(NSTEPS,),
        in_specs=[
            pl.BlockSpec((BB, L1), lambda i: (i, 0)),
            full((128, E)), full((128, E)), full((1, E)),
            full((L1, E * L2)), full((1, E * L2)),
            full((L1, E)), full((1, E)),
            full((E * L2, E * L3)), full((E * L2, E * L3)),
            full((1, E * L3)), full((E * L3, E)), full((1, E)),
        ],
        out_specs=(
            pl.BlockSpec((BB, 1), lambda i: (i, 0)),
            full((1, E)), full((1, E)), full((1, 1)), full((1, 1)),
            full((1, 1)), full((1, 1)), full((1, 1)),
        ),
        out_shape=out_shapes,
        scratch_shapes=[pltpu.VMEM((8, E), jnp.float32)],
        compiler_params=pltpu.CompilerParams(
            dimension_semantics=("arbitrary",)),
    )(x, rwa, rwb, rb, w1a, b1a, w1b, b1b, bda, bdb, b2, sw, ob)

    l3x, frac, avg, aux, z, ent, top1, std = outs
    return (l3x, aux[0, 0], z[0, 0], frac[0], avg[0],
            ent[0, 0], top1[0, 0], std[0, 0])


# speedup vs baseline: 1.0510x; 1.0351x over previous
import functools

import jax
import jax.numpy as jnp
import numpy as np
from jax.experimental import pallas as pl
from jax.experimental.pallas import tpu as pltpu

B = 8192
L1 = 2048
L2 = 16
L3 = 32
E = 16
RF = 32
BB = 1024
NSTEPS = B // BB


def _fused_kernel(x_ref, rwa_ref, rwb_ref, rb_ref, w1a_ref, b1a_ref,
                  w1b_ref, b1b_ref, bda_ref, bdb_ref, b2_ref, sw_ref, ob_ref,
                  l3x_ref, st_ref, acc_ref):
    i = pl.program_id(0)
    xb = x_ref[...]

    logits = (
        jnp.dot(xb[:, :128], rwa_ref[...], preferred_element_type=jnp.float32)
        + jnp.dot(xb[:, 1024:1152], rwb_ref[...],
                  preferred_element_type=jnp.float32)
        + rb_ref[...]
    )
    m = jnp.max(logits, axis=1, keepdims=True)
    ex = jnp.exp(logits - m)
    se = jnp.sum(ex, axis=1, keepdims=True)
    probs = ex / se
    lse = jnp.log(se) + m

    iota = jax.lax.broadcasted_iota(jnp.int32, logits.shape, 1)
    idx = jnp.min(jnp.where(logits == m, iota, E), axis=1, keepdims=True)
    onehot = (iota == idx).astype(jnp.float32)

    probs_sum = jnp.sum(probs, axis=0, keepdims=True)
    counts = jnp.sum(onehot, axis=0, keepdims=True)
    z_part = jnp.sum(lse * lse, axis=0, keepdims=True)
    ent_tok = jnp.sum(-(probs * jnp.log(jnp.maximum(probs, 1e-9))),
                      axis=1, keepdims=True)
    ent_part = jnp.sum(ent_tok, axis=0, keepdims=True)
    top1_part = jnp.sum(jnp.max(probs, axis=1, keepdims=True),
                        axis=0, keepdims=True)

    l1a = jnp.dot(xb, w1a_ref[...],
                  preferred_element_type=jnp.float32) + b1a_ref[...]
    l1o = jnp.dot(xb, w1b_ref[...],
                  preferred_element_type=jnp.float32) + b1b_ref[...]
    sq = jnp.clip(l1a * l1a * (255.0 / 256.0), 0.0, 1.0)
    lin = jnp.clip(l1a, 0.0, 1.0)
    b2 = b2_ref[...]
    bda = bda_ref[...]
    bdb = bdb_ref[...]
    sw = sw_ref[...]
    l2x0 = jnp.clip(
        jnp.dot(sq[:, :128], bda[:128, :256], preferred_element_type=jnp.float32)
        + jnp.dot(lin[:, :128], bdb[:128, :256], preferred_element_type=jnp.float32)
        + b2[:, :256], 0.0, 1.0)
    l2x1 = jnp.clip(
        jnp.dot(sq[:, 128:], bda[128:, 256:], preferred_element_type=jnp.float32)
        + jnp.dot(lin[:, 128:], bdb[128:, 256:], preferred_element_type=jnp.float32)
        + b2[:, 256:], 0.0, 1.0)
    y = (jnp.dot(l2x0, sw[:256, :], preferred_element_type=jnp.float32)
         + jnp.dot(l2x1, sw[256:, :], preferred_element_type=jnp.float32)
         + ob_ref[...] + l1o)

    mean_e = jnp.mean(y, axis=1, keepdims=True)
    var = jnp.mean(y * y, axis=1, keepdims=True) - mean_e * mean_e
    stdv = jnp.sqrt(jnp.maximum(var, 0.0))
    std_part = jnp.sum(stdv, axis=0, keepdims=True)

    l3x_ref[...] = jnp.sum(y * onehot, axis=1, keepdims=True)

    bc = lambda v: jnp.broadcast_to(v, (1, E))
    part = jnp.concatenate(
        [counts, probs_sum, bc(z_part), bc(ent_part), bc(top1_part),
         bc(std_part), jnp.zeros((2, E), jnp.float32)], axis=0)

    @pl.when(i == 0)
    def _init():
        acc_ref[...] = part

    @pl.when(i > 0)
    def _acc():
        acc_ref[...] += part

    @pl.when(i == NSTEPS - 1)
    def _finalize():
        st = acc_ref[...] / float(B)
        ri = jax.lax.broadcasted_iota(jnp.int32, (8, 1), 0)
        st = jnp.where(ri == 3, st / float(np.log(E)), st)
        aux = float(E) * jnp.sum(st[0:1, :] * st[1:2, :],
                                 axis=1, keepdims=True)
        st_ref[...] = jnp.where(ri == 6, jnp.broadcast_to(aux, (8, E)), st)


@jax.jit
def kernel(x, router_w, router_b, l1_w, l1_b, l2_w, l2_b, out_w, out_b):
    f32 = jnp.float32
    eye = jnp.eye(E, dtype=f32)

    rwa = jnp.zeros((128, E), f32).at[:RF, :].set(router_w[:, :RF].T)
    rwb = jnp.zeros((128, E), f32).at[:RF, :].set(router_w[:, RF:].T)
    rb = router_b.reshape(1, E)

    w1a = l1_w[:, :L2, :].reshape(E * L2, L1).T
    b1a = l1_b[:, :L2].reshape(1, E * L2)
    w1b = l1_w[:, L2, :].T
    b1b = l1_b[:, L2].reshape(1, E)

    bda = jnp.einsum('ef,eoi->eifo', eye,
                     l2_w[:, :, :L2]).reshape(E * L2, E * L3)
    bdb = jnp.einsum('ef,eoi->eifo', eye,
                     l2_w[:, :, L2:]).reshape(E * L2, E * L3)
    b2 = l2_b.reshape(1, E * L3)

    sw = jnp.einsum('ef,eo->eof', eye, out_w[:, 0, :]).reshape(E * L3, E)
    ob = out_b[:, 0].reshape(1, E)

    full = lambda shape: pl.BlockSpec(shape, lambda i: (0, 0))
    out_shapes = (
        jax.ShapeDtypeStruct((B, 1), f32),
        jax.ShapeDtypeStruct((8, E), f32),
    )
    outs = pl.pallas_call(
        _fused_kernel,
        grid=(NSTEPS,),
        in_specs=[
            pl.BlockSpec((BB, L1), lambda i: (i, 0)),
            full((128, E)), full((128, E)), full((1, E)),
            full((L1, E * L2)), full((1, E * L2)),
            full((L1, E)), full((1, E)),
            full((E * L2, E * L3)), full((E * L2, E * L3)),
            full((1, E * L3)), full((E * L3, E)), full((1, E)),
        ],
        out_specs=(
            pl.BlockSpec((BB, 1), lambda i: (i, 0)),
            full((8, E)),
        ),
        out_shape=out_shapes,
        scratch_shapes=[pltpu.VMEM((8, E), jnp.float32)],
        compiler_params=pltpu.CompilerParams(
            dimension_semantics=("arbitrary",)),
    )(x, rwa, rwb, rb, w1a, b1a, w1b, b1b, bda, bdb, b2, sw, ob)

    l3x, st = outs
    return (l3x, st[6, 0], st[2, 0], st[0], st[1],
            st[3, 0], st[4, 0], st[5, 0])


# speedup vs baseline: 1.0815x; 1.0289x over previous
import functools

import jax
import jax.numpy as jnp
import numpy as np
from jax.experimental import pallas as pl
from jax.experimental.pallas import tpu as pltpu

B = 8192
L1 = 2048
L2 = 16
L3 = 32
E = 16
RF = 32
BB = 1024
NSTEPS = B // BB


def _fused_kernel(x_ref, rwa_ref, rwb_ref, rb_ref, w1a_ref, b1a_ref,
                  w1b_ref, b1b_ref, bda_ref, bdb_ref, b2_ref, sw_ref, ob_ref,
                  l3x_ref, frac_ref, avg_ref, aux_ref, z_ref, ent_ref,
                  top1_ref, std_ref, acc_ref):
    i = pl.program_id(0)

    def _half(xb):
        logits = (
            jnp.dot(xb[:, :128], rwa_ref[...],
                    preferred_element_type=jnp.float32)
            + jnp.dot(xb[:, 1024:1152], rwb_ref[...],
                      preferred_element_type=jnp.float32)
            + rb_ref[...]
        )
        m = jnp.max(logits, axis=1, keepdims=True)
        ex = jnp.exp(logits - m)
        se = jnp.sum(ex, axis=1, keepdims=True)
        probs = ex / se
        lse = jnp.log(se) + m

        iota = jax.lax.broadcasted_iota(jnp.int32, logits.shape, 1)
        idx = jnp.min(jnp.where(logits == m, iota, E), axis=1, keepdims=True)
        onehot = (iota == idx).astype(jnp.float32)

        probs_sum = jnp.sum(probs, axis=0, keepdims=True)
        counts = jnp.sum(onehot, axis=0, keepdims=True)
        z_part = jnp.sum(lse * lse, axis=0, keepdims=True)
        ent_tok = jnp.sum(-(probs * jnp.log(jnp.maximum(probs, 1e-9))),
                          axis=1, keepdims=True)
        ent_part = jnp.sum(ent_tok, axis=0, keepdims=True)
        top1_part = jnp.sum(jnp.max(probs, axis=1, keepdims=True),
                            axis=0, keepdims=True)

        l1a = jnp.dot(xb, w1a_ref[...],
                      preferred_element_type=jnp.float32) + b1a_ref[...]
        l1o = jnp.dot(xb, w1b_ref[...],
                      preferred_element_type=jnp.float32) + b1b_ref[...]
        sq = jnp.clip(l1a * l1a * (255.0 / 256.0), 0.0, 1.0)
        lin = jnp.clip(l1a, 0.0, 1.0)
        b2 = b2_ref[...]
        bda = bda_ref[...]
        bdb = bdb_ref[...]
        sw = sw_ref[...]
        l2x0 = jnp.clip(
            jnp.dot(sq[:, :128], bda[:128, :256],
                    preferred_element_type=jnp.float32)
            + jnp.dot(lin[:, :128], bdb[:128, :256],
                      preferred_element_type=jnp.float32)
            + b2[:, :256], 0.0, 1.0)
        l2x1 = jnp.clip(
            jnp.dot(sq[:, 128:], bda[128:, 256:],
                    preferred_element_type=jnp.float32)
            + jnp.dot(lin[:, 128:], bdb[128:, 256:],
                      preferred_element_type=jnp.float32)
            + b2[:, 256:], 0.0, 1.0)
        y = (jnp.dot(l2x0, sw[:256, :], preferred_element_type=jnp.float32)
             + jnp.dot(l2x1, sw[256:, :], preferred_element_type=jnp.float32)
             + ob_ref[...] + l1o)

        mean_e = jnp.mean(y, axis=1, keepdims=True)
        var = jnp.mean(y * y, axis=1, keepdims=True) - mean_e * mean_e
        stdv = jnp.sqrt(jnp.maximum(var, 0.0))
        std_part = jnp.sum(stdv, axis=0, keepdims=True)

        l3x = jnp.sum(y * onehot, axis=1, keepdims=True)
        bc = lambda v: jnp.broadcast_to(v, (1, E))
        part = jnp.concatenate(
            [counts, probs_sum, bc(z_part), bc(ent_part), bc(top1_part),
             bc(std_part), jnp.zeros((2, E), jnp.float32)], axis=0)
        return l3x, part

    xfull = x_ref[...]
    l3xa, parta = _half(xfull[:BB // 2])
    l3xb, partb = _half(xfull[BB // 2:])
    l3x_ref[:BB // 2, :] = l3xa
    l3x_ref[BB // 2:, :] = l3xb
    part = parta + partb

    @pl.when(i == 0)
    def _init():
        acc_ref[...] = part

    @pl.when(i > 0)
    def _acc():
        acc_ref[...] += part

    @pl.when(i == NSTEPS - 1)
    def _finalize():
        st = acc_ref[...] / float(B)
        fr = st[0:1, :]
        av = st[1:2, :]
        frac_ref[...] = fr
        avg_ref[...] = av
        aux_ref[...] = float(E) * jnp.sum(fr * av, axis=1, keepdims=True)
        z_ref[...] = st[2:3, 0:1]
        ent_ref[...] = st[3:4, 0:1] / float(np.log(E))
        top1_ref[...] = st[4:5, 0:1]
        std_ref[...] = st[5:6, 0:1]


@jax.jit
def kernel(x, router_w, router_b, l1_w, l1_b, l2_w, l2_b, out_w, out_b):
    f32 = jnp.float32
    eye = jnp.eye(E, dtype=f32)

    rwa = jnp.zeros((128, E), f32).at[:RF, :].set(router_w[:, :RF].T)
    rwb = jnp.zeros((128, E), f32).at[:RF, :].set(router_w[:, RF:].T)
    rb = router_b.reshape(1, E)

    w1a = l1_w[:, :L2, :].reshape(E * L2, L1).T
    b1a = l1_b[:, :L2].reshape(1, E * L2)
    w1b = l1_w[:, L2, :].T
    b1b = l1_b[:, L2].reshape(1, E)

    bda = jnp.einsum('ef,eoi->eifo', eye,
                     l2_w[:, :, :L2]).reshape(E * L2, E * L3)
    bdb = jnp.einsum('ef,eoi->eifo', eye,
                     l2_w[:, :, L2:]).reshape(E * L2, E * L3)
    b2 = l2_b.reshape(1, E * L3)

    sw = jnp.einsum('ef,eo->eof', eye, out_w[:, 0, :]).reshape(E * L3, E)
    ob = out_b[:, 0].reshape(1, E)

    full = lambda shape: pl.BlockSpec(shape, lambda i: (0, 0))
    out_shapes = (
        jax.ShapeDtypeStruct((B, 1), f32),
        jax.ShapeDtypeStruct((1, E), f32),
        jax.ShapeDtypeStruct((1, E), f32),
        jax.ShapeDtypeStruct((1, 1), f32),
        jax.ShapeDtypeStruct((1, 1), f32),
        jax.ShapeDtypeStruct((1, 1), f32),
        jax.ShapeDtypeStruct((1, 1), f32),
        jax.ShapeDtypeStruct((1, 1), f32),
    )
    outs = pl.pallas_call(
        _fused_kernel,
        grid=(NSTEPS,),
        in_specs=[
            pl.BlockSpec((BB, L1), lambda i: (i, 0)),
            full((128, E)), full((128, E)), full((1, E)),
            full((L1, E * L2)), full((1, E * L2)),
            full((L1, E)), full((1, E)),
            full((E * L2, E * L3)), full((E * L2, E * L3)),
            full((1, E * L3)), full((E * L3, E)), full((1, E)),
        ],
        out_specs=(
            pl.BlockSpec((BB, 1), lambda i: (i, 0)),
            full((1, E)), full((1, E)), full((1, 1)), full((1, 1)),
            full((1, 1)), full((1, 1)), full((1, 1)),
        ),
        out_shape=out_shapes,
        scratch_shapes=[pltpu.VMEM((8, E), jnp.float32)],
        compiler_params=pltpu.CompilerParams(
            dimension_semantics=("arbitrary",)),
    )(x, rwa, rwb, rb, w1a, b1a, w1b, b1b, bda, bdb, b2, sw, ob)

    l3x, frac, avg, aux, z, ent, top1, std = outs
    return (l3x, aux[0, 0], z[0, 0], frac[0], avg[0],
            ent[0, 0], top1[0, 0], std[0, 0])
